# baseline (speedup 1.0000x reference)
import functools

import jax
import jax.numpy as jnp
from jax import lax
from jax.experimental import pallas as pl
from jax.experimental.pallas import tpu as pltpu
from jax.experimental.pallas import tpu_sc as plsc


def _make_sc_broadcast(B: int, S: int, D: int, dtype):
    info = plsc.get_sparse_core_info()
    NC, NS = info.num_cores, info.num_subcores
    NW = NC * NS
    assert S % NW == 0
    rows_per_w = S // NW
    chunk = min(32, rows_per_w)
    assert rows_per_w % chunk == 0
    n_chunks = rows_per_w // chunk

    mesh = plsc.VectorSubcoreMesh(core_axis_name="c", subcore_axis_name="s")

    @functools.partial(
        pl.kernel,
        mesh=mesh,
        out_type=jax.ShapeDtypeStruct((B, S, D), dtype),
        scratch_types=[
            pltpu.VMEM((chunk, D), dtype),
            pltpu.SemaphoreType.DMA,
        ],
    )
    def broadcast_rows(table_hbm, out_hbm, buf, wsem):
        wid = lax.axis_index("s") * NC + lax.axis_index("c")
        base = wid * rows_per_w
        for j in range(n_chunks):
            r0 = base + j * chunk
            pltpu.sync_copy(table_hbm.at[pl.ds(r0, chunk), :], buf)
            cps = []
            for b in range(B):
                cp = pltpu.make_async_copy(
                    buf, out_hbm.at[b, pl.ds(r0, chunk), :], wsem)
                cp.start()
                cps.append(cp)
            for cp in cps:
                cp.wait()

    return broadcast_rows


def _make_tc_broadcast(B: int, S: int, D: int, dtype):
    bs = 256
    assert S % bs == 0

    def body(in_ref, out_ref):
        out_ref[...] = jnp.broadcast_to(in_ref[...][None], (B, bs, D))

    return pl.pallas_call(
        body,
        grid=(S // bs,),
        in_specs=[pl.BlockSpec((bs, D), lambda i: (i, 0))],
        out_specs=pl.BlockSpec((B, bs, D), lambda i: (0, i, 0)),
        out_shape=jax.ShapeDtypeStruct((B, S, D), dtype),
    )


def kernel(x, position_embedding):
    B, S, _ = x.shape
    _, D = position_embedding.shape
    Bsc = B // 2
    Btc = B - Bsc
    sc = _make_sc_broadcast(Bsc, S, D, position_embedding.dtype)
    tc = _make_tc_broadcast(Btc, S, D, position_embedding.dtype)
    return jnp.concatenate([sc(position_embedding), tc(position_embedding)],
                           axis=0)


# speedup vs baseline: 2.3480x; 2.3480x over previous
import functools

import jax
import jax.numpy as jnp
from jax import lax
from jax.experimental import pallas as pl
from jax.experimental.pallas import tpu as pltpu
from jax.experimental.pallas import tpu_sc as plsc


def _make_sc_broadcast(B: int, S: int, D: int, dtype):
    info = plsc.get_sparse_core_info()
    NC, NS = info.num_cores, info.num_subcores
    NW = NC * NS
    assert S % NW == 0
    rows_per_w = S // NW
    chunk = min(rows_per_w, max(8, (131071 // D) & ~7))
    n_full, rem = divmod(rows_per_w, chunk)
    chunks = [chunk] * n_full + ([rem] if rem else [])
    offs = [i * chunk for i in range(len(chunks))]

    mesh = plsc.VectorSubcoreMesh(core_axis_name="c", subcore_axis_name="s")

    @functools.partial(
        pl.kernel,
        mesh=mesh,
        out_type=jax.ShapeDtypeStruct((B, S, D), dtype),
        scratch_types=[
            pltpu.VMEM((chunk, D), dtype),
            pltpu.SemaphoreType.DMA,
        ],
    )
    def broadcast_rows(table_hbm, out_hbm, buf, wsem):
        wid = lax.axis_index("s") * NC + lax.axis_index("c")
        base = wid * rows_per_w
        for j, c in enumerate(chunks):
            r0 = base + offs[j]
            src = buf if c == chunk else buf.at[pl.ds(0, c), :]
            pltpu.sync_copy(table_hbm.at[pl.ds(r0, c), :], src)
            cps = []
            for b in range(B):
                cp = pltpu.make_async_copy(
                    src, out_hbm.at[b, pl.ds(r0, c), :], wsem)
                cp.start()
                cps.append(cp)
            for cp in cps:
                cp.wait()

    return broadcast_rows


def kernel(x, position_embedding):
    B, S, _ = x.shape
    _, D = position_embedding.shape
    fn = _make_sc_broadcast(B, S, D, position_embedding.dtype)
    return fn(position_embedding)
